# baseline (speedup 1.0000x reference)
import functools

import jax
import jax.numpy as jnp
from jax import lax
from jax.experimental import pallas as pl
from jax.experimental.pallas import tpu as pltpu
from jax.experimental.pallas import tpu_sc as plsc

N, K, T, H = 32, 16, 128, 1024
NEGINF = -1e20
NC, NS, L = 2, 16, 16

HB = 128
NHB = H // HB

GROUPS = (64, 56, 8)
OFFS = (0, 64, 120)


def _score_body(qcat_ref, wcqk_ref, bcqk_ref, wk_ref, bk_ref, ckm_ref,
                pool1_ref, score_ref, cq_ref):
    j = pl.program_id(0)

    @pl.when(j < NHB)
    def _():
        cq_ref[:, pl.ds(j * HB, HB)] = lax.dot_general(
            qcat_ref[...], wcqk_ref[...], (((1,), (1,)), ((), ())),
            preferred_element_type=jnp.float32) + bcqk_ref[...]

    @pl.when(j >= NHB)
    def _():
        jj = j - NHB
        cq = cq_ref[...]
        t = jnp.dot(cq.astype(jnp.bfloat16), wk_ref[...],
                    preferred_element_type=jnp.float32)
        part = jnp.sum(pool1_ref[...] * t[:, None, :], axis=2)

        @pl.when(jj == 0)
        def _():
            bias = jnp.sum(cq * bk_ref[...], axis=1, keepdims=True)
            score_ref[...] = part + bias

        @pl.when(jj > 0)
        def _():
            score_ref[...] += part

        @pl.when(jj == NHB - 1)
        def _():
            score_ref[...] = jnp.where(ckm_ref[...] != 0, score_ref[...],
                                       NEGINF)


def _gather_body(rows_hbm, pool0_hbm, pool1_hbm, mask_hbm, pidx_hbm,
                 enc_hbm, use_hbm, masko_hbm, pidxo_hbm,
                 rows_v, buf0, buf1, buf_use, buf_mask, buf_pidx,
                 sin0, sin1, sin2, sout0, sout1, sout2, ssm, ssm2):
    w = lax.axis_index("s") * NC + lax.axis_index("c")
    pltpu.sync_copy(rows_hbm, rows_v)
    splat = plsc.load_gather(rows_v, [jnp.full((L,), w, jnp.int32)])
    row = jnp.max(splat)

    sm_in = [
        pltpu.make_async_copy(pool1_hbm.at[pl.ds(row, 1)], buf_use, ssm),
        pltpu.make_async_copy(mask_hbm.at[pl.ds(row, 1)], buf_mask, ssm),
        pltpu.make_async_copy(pidx_hbm.at[pl.ds(row, 1)], buf_pidx, ssm),
    ]
    sm_out = [
        pltpu.make_async_copy(buf_use, use_hbm.at[pl.ds(w, 1)], ssm2),
        pltpu.make_async_copy(buf_mask, masko_hbm.at[pl.ds(w, 1)], ssm2),
        pltpu.make_async_copy(buf_pidx, pidxo_hbm.at[pl.ds(w, 1)], ssm2),
    ]
    for cp in sm_in:
        cp.start()

    bufs = (buf0, buf1, buf0.at[pl.ds(0, GROUPS[2])])
    sins, souts = (sin0, sin1, sin2), (sout0, sout1, sout2)
    ins = [pltpu.make_async_copy(
        pool0_hbm.at[pl.ds(row * T + OFFS[g], GROUPS[g])],
        bufs[g], sins[g]) for g in range(3)]
    outs = [pltpu.make_async_copy(
        bufs[g], enc_hbm.at[pl.ds(w * T + OFFS[g], GROUPS[g])],
        souts[g]) for g in range(3)]
    ins[0].start()
    ins[1].start()
    ins[0].wait()
    outs[0].start()
    ins[1].wait()
    outs[1].start()
    outs[0].wait()
    ins[2].start()
    ins[2].wait()
    outs[2].start()
    for cp in sm_in:
        cp.wait()
    for cp in sm_out:
        cp.start()
    outs[1].wait()
    outs[2].wait()
    for cp in sm_out:
        cp.wait()


def kernel(contexts_encoded_1, tracked_knowledge_use,
           knowledge_shifting_pool_encoded_0, knowledge_shifting_pool_encoded_1,
           knowledge_shifting_pool_mask, shifting_ck_mask,
           knowledge_shifting_label, knowledge_shifting_pool,
           W_cqk, b_cqk, W_k, b_k):
    ids = knowledge_shifting_label.astype(jnp.int32)
    rows = jnp.arange(N, dtype=jnp.int32) * K + ids

    qcat = jnp.concatenate(
        [contexts_encoded_1[:, 2, :], tracked_knowledge_use], axis=1)
    ckm = shifting_ck_mask.astype(jnp.int32)

    score = pl.pallas_call(
        _score_body,
        grid=(2 * NHB,),
        in_specs=[
            pl.BlockSpec((N, 2 * H), lambda j: (0, 0)),
            pl.BlockSpec((HB, 2 * H), lambda j: (jnp.minimum(j, NHB - 1), 0)),
            pl.BlockSpec((1, HB), lambda j: (0, jnp.minimum(j, NHB - 1))),
            pl.BlockSpec((H, HB), lambda j: (0, jnp.maximum(j - NHB, 0))),
            pl.BlockSpec((1, H), lambda j: (0, 0)),
            pl.BlockSpec((N, K), lambda j: (0, 0)),
            pl.BlockSpec((N, K, HB), lambda j: (0, 0, jnp.maximum(j - NHB, 0))),
        ],
        out_specs=pl.BlockSpec((N, K), lambda j: (0, 0)),
        scratch_shapes=[pltpu.VMEM((N, H), jnp.float32)],
        out_shape=jax.ShapeDtypeStruct((N, K), jnp.float32),
    )(qcat.astype(jnp.bfloat16), W_cqk.astype(jnp.bfloat16),
      b_cqk.reshape(1, H), W_k.astype(jnp.bfloat16), b_k.reshape(1, H), ckm,
      knowledge_shifting_pool_encoded_1)

    mesh = plsc.VectorSubcoreMesh(core_axis_name="c", subcore_axis_name="s")
    gather = functools.partial(
        pl.kernel,
        out_type=[
            jax.ShapeDtypeStruct((N * T, H), jnp.float32),
            jax.ShapeDtypeStruct((N, H), jnp.float32),
            jax.ShapeDtypeStruct((N, T), jnp.bool_),
            jax.ShapeDtypeStruct((N, T), jnp.int32),
        ],
        mesh=mesh,
        scratch_types=[
            pltpu.VMEM((N,), jnp.int32),
            pltpu.VMEM((GROUPS[0], H), jnp.float32),
            pltpu.VMEM((GROUPS[1], H), jnp.float32),
            pltpu.VMEM((1, H), jnp.float32),
            pltpu.VMEM((1, T), jnp.bool_),
            pltpu.VMEM((1, T), jnp.int32),
            pltpu.SemaphoreType.DMA,
            pltpu.SemaphoreType.DMA,
            pltpu.SemaphoreType.DMA,
            pltpu.SemaphoreType.DMA,
            pltpu.SemaphoreType.DMA,
            pltpu.SemaphoreType.DMA,
            pltpu.SemaphoreType.DMA,
            pltpu.SemaphoreType.DMA,
        ],
        compiler_params=pltpu.CompilerParams(needs_layout_passes=False),
    )(_gather_body)
    enc, use, masko, pidxo = gather(
        rows,
        knowledge_shifting_pool_encoded_0.reshape(N * K * T, H),
        knowledge_shifting_pool_encoded_1.reshape(N * K, H),
        knowledge_shifting_pool_mask.reshape(N * K, T),
        knowledge_shifting_pool.reshape(N * K, T),
    )

    return (score, enc.reshape(N, T, H), masko, use,
            pidxo.astype(knowledge_shifting_pool.dtype))


# speedup vs baseline: 1.1796x; 1.1796x over previous
import functools

import jax
import jax.numpy as jnp
from jax import lax
from jax.experimental import pallas as pl
from jax.experimental.pallas import tpu as pltpu
from jax.experimental.pallas import tpu_sc as plsc

N, K, T, H = 32, 16, 128, 1024
NEGINF = -1e20
NC, NS, L = 2, 16, 16

HB = 128
NHB = H // HB

GROUPS = (64, 56, 8)
OFFS = (0, 64, 120)


def _score_body(qcat_ref, wcqk_ref, bcqk_ref, wk_ref, bk_ref, ckm_ref,
                pool1_ref, score_ref, cq_ref):
    j = pl.program_id(0)

    @pl.when(j < NHB)
    def _():
        cq_ref[:, pl.ds(j * HB, HB)] = lax.dot_general(
            qcat_ref[...].astype(jnp.bfloat16),
            wcqk_ref[...].astype(jnp.bfloat16), (((1,), (1,)), ((), ())),
            preferred_element_type=jnp.float32) + bcqk_ref[...]

    @pl.when(j >= NHB)
    def _():
        jj = j - NHB
        cq = cq_ref[...]
        t = jnp.dot(cq.astype(jnp.bfloat16),
                    wk_ref[...].astype(jnp.bfloat16),
                    preferred_element_type=jnp.float32)
        part = jnp.sum(pool1_ref[...] * t[:, None, :], axis=2)

        @pl.when(jj == 0)
        def _():
            bias = jnp.sum(cq * bk_ref[...], axis=1, keepdims=True)
            score_ref[...] = part + bias

        @pl.when(jj > 0)
        def _():
            score_ref[...] += part

        @pl.when(jj == NHB - 1)
        def _():
            score_ref[...] = jnp.where(ckm_ref[...] != 0, score_ref[...],
                                       NEGINF)


def _gather_body(rows_hbm, pool0_hbm, pool1_hbm, mask_hbm, pidx_hbm,
                 enc_hbm, use_hbm, masko_hbm, pidxo_hbm,
                 rows_v, buf0, buf1, buf_use, buf_mask, buf_pidx,
                 sin0, sin1, sin2, sout0, sout1, sout2, ssm, ssm2):
    w = lax.axis_index("s") * NC + lax.axis_index("c")
    pltpu.sync_copy(rows_hbm, rows_v)
    splat = plsc.load_gather(rows_v, [jnp.full((L,), w, jnp.int32)])
    row = jnp.max(splat)

    sm_in = [
        pltpu.make_async_copy(pool1_hbm.at[pl.ds(row, 1)], buf_use, ssm),
        pltpu.make_async_copy(mask_hbm.at[pl.ds(row, 1)], buf_mask, ssm),
        pltpu.make_async_copy(pidx_hbm.at[pl.ds(row, 1)], buf_pidx, ssm),
    ]
    sm_out = [
        pltpu.make_async_copy(buf_use, use_hbm.at[pl.ds(w, 1)], ssm2),
        pltpu.make_async_copy(buf_mask, masko_hbm.at[pl.ds(w, 1)], ssm2),
        pltpu.make_async_copy(buf_pidx, pidxo_hbm.at[pl.ds(w, 1)], ssm2),
    ]
    for cp in sm_in:
        cp.start()

    bufs = (buf0, buf1, buf0.at[pl.ds(0, GROUPS[2])])
    sins, souts = (sin0, sin1, sin2), (sout0, sout1, sout2)
    ins = [pltpu.make_async_copy(
        pool0_hbm.at[pl.ds(row * T + OFFS[g], GROUPS[g])],
        bufs[g], sins[g]) for g in range(3)]
    outs = [pltpu.make_async_copy(
        bufs[g], enc_hbm.at[pl.ds(w * T + OFFS[g], GROUPS[g])],
        souts[g]) for g in range(3)]
    ins[0].start()
    ins[1].start()
    ins[0].wait()
    outs[0].start()
    ins[1].wait()
    outs[1].start()
    outs[0].wait()
    ins[2].start()
    ins[2].wait()
    outs[2].start()
    for cp in sm_in:
        cp.wait()
    for cp in sm_out:
        cp.start()
    outs[1].wait()
    outs[2].wait()
    for cp in sm_out:
        cp.wait()


def kernel(contexts_encoded_1, tracked_knowledge_use,
           knowledge_shifting_pool_encoded_0, knowledge_shifting_pool_encoded_1,
           knowledge_shifting_pool_mask, shifting_ck_mask,
           knowledge_shifting_label, knowledge_shifting_pool,
           W_cqk, b_cqk, W_k, b_k):
    ids = knowledge_shifting_label.astype(jnp.int32)
    rows = jnp.arange(N, dtype=jnp.int32) * K + ids

    qcat = jnp.concatenate(
        [contexts_encoded_1[:, 2, :], tracked_knowledge_use], axis=1)
    ckm = shifting_ck_mask.astype(jnp.int32)

    score = pl.pallas_call(
        _score_body,
        grid=(2 * NHB,),
        in_specs=[
            pl.BlockSpec((N, 2 * H), lambda j: (0, 0)),
            pl.BlockSpec((HB, 2 * H), lambda j: (jnp.minimum(j, NHB - 1), 0)),
            pl.BlockSpec((1, HB), lambda j: (0, jnp.minimum(j, NHB - 1))),
            pl.BlockSpec((H, HB), lambda j: (0, jnp.maximum(j - NHB, 0))),
            pl.BlockSpec((1, H), lambda j: (0, 0)),
            pl.BlockSpec((N, K), lambda j: (0, 0)),
            pl.BlockSpec((N, K, HB), lambda j: (0, 0, jnp.maximum(j - NHB, 0))),
        ],
        out_specs=pl.BlockSpec((N, K), lambda j: (0, 0)),
        scratch_shapes=[pltpu.VMEM((N, H), jnp.float32)],
        out_shape=jax.ShapeDtypeStruct((N, K), jnp.float32),
    )(qcat, W_cqk, b_cqk.reshape(1, H), W_k, b_k.reshape(1, H), ckm,
      knowledge_shifting_pool_encoded_1)

    mesh = plsc.VectorSubcoreMesh(core_axis_name="c", subcore_axis_name="s")
    gather = functools.partial(
        pl.kernel,
        out_type=[
            jax.ShapeDtypeStruct((N * T, H), jnp.float32),
            jax.ShapeDtypeStruct((N, H), jnp.float32),
            jax.ShapeDtypeStruct((N, T), jnp.bool_),
            jax.ShapeDtypeStruct((N, T), jnp.int32),
        ],
        mesh=mesh,
        scratch_types=[
            pltpu.VMEM((N,), jnp.int32),
            pltpu.VMEM((GROUPS[0], H), jnp.float32),
            pltpu.VMEM((GROUPS[1], H), jnp.float32),
            pltpu.VMEM((1, H), jnp.float32),
            pltpu.VMEM((1, T), jnp.bool_),
            pltpu.VMEM((1, T), jnp.int32),
            pltpu.SemaphoreType.DMA,
            pltpu.SemaphoreType.DMA,
            pltpu.SemaphoreType.DMA,
            pltpu.SemaphoreType.DMA,
            pltpu.SemaphoreType.DMA,
            pltpu.SemaphoreType.DMA,
            pltpu.SemaphoreType.DMA,
            pltpu.SemaphoreType.DMA,
        ],
        compiler_params=pltpu.CompilerParams(needs_layout_passes=False),
    )(_gather_body)
    enc, use, masko, pidxo = gather(
        rows,
        knowledge_shifting_pool_encoded_0.reshape(N * K * T, H),
        knowledge_shifting_pool_encoded_1.reshape(N * K, H),
        knowledge_shifting_pool_mask.reshape(N * K, T),
        knowledge_shifting_pool.reshape(N * K, T),
    )

    return (score, enc.reshape(N, T, H), masko, use,
            pidxo.astype(knowledge_shifting_pool.dtype))
